# baseline (speedup 1.0000x reference)
import functools

import jax
import jax.numpy as jnp
from jax import lax
from jax.experimental import pallas as pl
from jax.experimental.pallas import tpu as pltpu
from jax.experimental.pallas import tpu_sc as plsc

N = 50000
E = 800000
B = 512
HID = 32
LAYERS = 5
EPS = 1e-5
NODE_W = [12, 3, 3, 3, 3, 3, 3, 3, 2, 2]
EDGE_W = [8, 3, 2, 2, 2]
NF = 37
EF = 17
NEG_SLOPE = 0.2

NCORES = 2
NSUB = 16
NW = NCORES * NSUB
CH = 128
E_PC = 25088
E_PAD = NW * E_PC
NCHUNK = E_PC // CH
ROWS_PT = 3128
N_ACC = NSUB * ROWS_PT
ACC_W = HID + 1


def _edge_pass_body(src_hbm, dst_hbm, aer_hbm, asrc_hbm, adst_hbm, hw_hbm,
                    m_hbm, zeros_hbm, out_hbm,
                    acc_sh, asrc_sh, adst_sh, srcb, dstb, aerb, abuf, bbuf,
                    tb, rows, stag, mv, sem):
    c = lax.axis_index("c")
    s = lax.axis_index("s")
    wid = s * NCORES + c

    pltpu.sync_copy(zeros_hbm, acc_sh.at[pl.ds(s * ROWS_PT, ROWS_PT)])

    @pl.when(s == 0)
    def _():
        pltpu.sync_copy(asrc_hbm, asrc_sh)
        pltpu.sync_copy(adst_hbm, adst_sh)

    pltpu.sync_copy(m_hbm, mv)
    plsc.subcore_barrier()

    base = wid * E_PC
    mvec = mv[...]

    iota16 = lax.iota(jnp.int32, 16)
    col32 = jnp.full((16,), HID, jnp.int32)

    def chunk(i, carry):
        off = base + i * CH
        pltpu.sync_copy(src_hbm.at[pl.ds(off, CH)], srcb)
        pltpu.sync_copy(dst_hbm.at[pl.ds(off, CH)], dstb)
        pltpu.sync_copy(aer_hbm.at[pl.ds(off, CH)], aerb)
        pltpu.async_copy(asrc_sh.at[srcb], abuf, sem).wait()
        pltpu.async_copy(adst_sh.at[dstb], bbuf, sem).wait()
        pltpu.async_copy(hw_hbm.at[srcb], rows, sem).wait()
        for j in range(CH // 16):
            a = abuf[pl.ds(j * 16, 16)]
            bb = bbuf[pl.ds(j * 16, 16)]
            z = a + bb + aerb[pl.ds(j * 16, 16)]
            z = jnp.where(z >= 0, z, z * NEG_SLOPE)
            t = jnp.exp(z - mvec)
            tb[pl.ds(j * 16, 16)] = t
            plsc.store_scatter(stag, [iota16 + j * 16, col32], t)
            for k in range(16):
                e = j * 16 + k
                ts = plsc.load_gather(tb, [jnp.full((16,), e, jnp.int32)])
                stag[e, pl.ds(0, 16)] = rows[e, pl.ds(0, 16)] * ts
                stag[e, pl.ds(16, 16)] = rows[e, pl.ds(16, 16)] * ts
        pltpu.sync_copy(stag, acc_sh.at[dstb], add=True)
        return carry

    lax.fori_loop(0, NCHUNK, chunk, 0)
    plsc.subcore_barrier()
    pltpu.sync_copy(acc_sh.at[pl.ds(s * ROWS_PT, ROWS_PT)],
                    out_hbm.at[pl.ds(c * N_ACC + s * ROWS_PT, ROWS_PT)])


_edge_pass = functools.partial(
    pl.kernel,
    out_type=jax.ShapeDtypeStruct((NCORES * N_ACC, ACC_W), jnp.float32),
    mesh=plsc.VectorSubcoreMesh(core_axis_name="c", subcore_axis_name="s"),
    compiler_params=pltpu.CompilerParams(needs_layout_passes=False,
                                         use_tc_tiling_on_sc=False),
    scratch_types=[
        pltpu.VMEM_SHARED((N_ACC, ACC_W), jnp.float32),
        pltpu.VMEM_SHARED((N,), jnp.float32),
        pltpu.VMEM_SHARED((N,), jnp.float32),
        pltpu.VMEM((CH,), jnp.int32),
        pltpu.VMEM((CH,), jnp.int32),
        pltpu.VMEM((CH,), jnp.float32),
        pltpu.VMEM((CH,), jnp.float32),
        pltpu.VMEM((CH,), jnp.float32),
        pltpu.VMEM((CH,), jnp.float32),
        pltpu.VMEM((CH, HID), jnp.float32),
        pltpu.VMEM((CH, ACC_W), jnp.float32),
        pltpu.VMEM((16,), jnp.float32),
        pltpu.SemaphoreType.DMA,
    ],
)(_edge_pass_body)


RO_BLK = 1000
RO_GRID = N // RO_BLK
NBLK = N // RO_BLK
EBLK = 8000
EGRID = E // EBLK


def _onehot3(ints):
    return jnp.concatenate(
        [(ints == k).astype(jnp.float32) for k in range(3)], axis=1)


def _stats_body(ints_ref, cnt_ref):
    @pl.when(pl.program_id(0) == 0)
    def _():
        cnt_ref[...] = jnp.zeros_like(cnt_ref)
    oh = _onehot3(ints_ref[0])
    cnt_ref[...] += jnp.sum(oh, axis=0, keepdims=True)


def _stats_call(ints3d, nblk, blk, ncols):
    return pl.pallas_call(
        _stats_body,
        grid=(nblk,),
        in_specs=[pl.BlockSpec((1, blk, ncols), lambda i: (i, 0, 0))],
        out_specs=pl.BlockSpec((1, 3 * ncols), lambda i: (0, 0)),
        out_shape=jax.ShapeDtypeStruct((1, 3 * ncols), jnp.float32),
    )(ints3d)


def _prep_int_body(ints_ref, w_ref, c_ref, hw_ref, asrc_ref, adst_ref, mx_ref):
    @pl.when(pl.program_id(0) == 0)
    def _():
        mx_ref[...] = jnp.full_like(mx_ref, -1e30)
    oh = _onehot3(ints_ref[0])
    out = jnp.dot(oh, w_ref[...], preferred_element_type=jnp.float32) + c_ref[...]
    hw_ref[...] = out[:, :HID]
    asrc_ref[...] = out[:, HID:HID + 1]
    adst_ref[...] = out[:, HID + 1:HID + 2]
    mx_ref[...] = jnp.maximum(mx_ref[...], jnp.max(out, axis=0, keepdims=True))


def _prep_f32_body(h_ref, w_ref, c_ref, hw_ref, asrc_ref, adst_ref, mx_ref):
    @pl.when(pl.program_id(0) == 0)
    def _():
        mx_ref[...] = jnp.full_like(mx_ref, -1e30)
    out = (jnp.dot(h_ref[...], w_ref[...], preferred_element_type=jnp.float32)
           + c_ref[...])
    hw_ref[...] = out[:, :HID]
    asrc_ref[...] = out[:, HID:HID + 1]
    adst_ref[...] = out[:, HID + 1:HID + 2]
    mx_ref[...] = jnp.maximum(mx_ref[...], jnp.max(out, axis=0, keepdims=True))


_PREP_OUT_SPECS = [
    pl.BlockSpec((RO_BLK, HID), lambda i: (i, 0)),
    pl.BlockSpec((RO_BLK, 1), lambda i: (i, 0)),
    pl.BlockSpec((RO_BLK, 1), lambda i: (i, 0)),
    pl.BlockSpec((1, 34), lambda i: (0, 0)),
]
_PREP_OUT_SHAPE = [
    jax.ShapeDtypeStruct((N, HID), jnp.float32),
    jax.ShapeDtypeStruct((N, 1), jnp.float32),
    jax.ShapeDtypeStruct((N, 1), jnp.float32),
    jax.ShapeDtypeStruct((1, 34), jnp.float32),
]


def _prep_int(ints3d, w, crow):
    kdim, odim = w.shape
    return pl.pallas_call(
        _prep_int_body,
        grid=(NBLK,),
        in_specs=[
            pl.BlockSpec((1, RO_BLK, kdim // 3), lambda i: (i, 0, 0)),
            pl.BlockSpec((kdim, odim), lambda i: (0, 0)),
            pl.BlockSpec((1, odim), lambda i: (0, 0)),
        ],
        out_specs=_PREP_OUT_SPECS,
        out_shape=_PREP_OUT_SHAPE,
    )(ints3d, w, crow)


def _prep_f32(h, w, crow):
    kdim, odim = w.shape
    return pl.pallas_call(
        _prep_f32_body,
        grid=(NBLK,),
        in_specs=[
            pl.BlockSpec((RO_BLK, kdim), lambda i: (i, 0)),
            pl.BlockSpec((kdim, odim), lambda i: (0, 0)),
            pl.BlockSpec((1, odim), lambda i: (0, 0)),
        ],
        out_specs=_PREP_OUT_SPECS,
        out_shape=_PREP_OUT_SHAPE,
    )(h, w, crow)


def _aer_body(ints_ref, w_ref, c_ref, out_ref, mx_ref):
    @pl.when(pl.program_id(0) == 0)
    def _():
        mx_ref[...] = jnp.full_like(mx_ref, -1e30)
    oh = _onehot3(ints_ref[0])
    out = jnp.dot(oh, w_ref[...], preferred_element_type=jnp.float32) + c_ref[...]
    out_ref[...] = out
    mx_ref[...] = jnp.maximum(mx_ref[...], jnp.max(out, axis=0, keepdims=True))


def _aer_call(ints3d, w, crow):
    return pl.pallas_call(
        _aer_body,
        grid=(EGRID,),
        in_specs=[
            pl.BlockSpec((1, EBLK, 5), lambda i: (i, 0, 0)),
            pl.BlockSpec((15, LAYERS), lambda i: (0, 0)),
            pl.BlockSpec((1, LAYERS), lambda i: (0, 0)),
        ],
        out_specs=[
            pl.BlockSpec((EBLK, LAYERS), lambda i: (i, 0)),
            pl.BlockSpec((1, LAYERS), lambda i: (0, 0)),
        ],
        out_shape=[
            jax.ShapeDtypeStruct((E, LAYERS), jnp.float32),
            jax.ShapeDtypeStruct((1, LAYERS), jnp.float32),
        ],
    )(ints3d, w, crow)


def _make_finish_body(with_relu):
    def _finish_body(acca_ref, accb_ref, hw_ref, asrc_ref, adst_ref,
                     scal_ref, bias_ref, out_ref):
        asd = asrc_ref[...] + adst_ref[...]
        z = asd + scal_ref[0, 1]
        z = jnp.where(z >= 0, z, z * NEG_SLOPE)
        ts = jnp.exp(z - scal_ref[0, 0])
        num = (acca_ref[:, :HID] + accb_ref[:, :HID]
               + ts * hw_ref[...])
        den = acca_ref[:, HID:HID + 1] + accb_ref[:, HID:HID + 1] + ts
        h = num / den + bias_ref[...]
        out_ref[...] = jnp.maximum(h, 0.0) if with_relu else h
    return _finish_body


def _finish(acca, accb, hw, asrc, adst, scal, bias_row, with_relu):
    return pl.pallas_call(
        _make_finish_body(with_relu),
        grid=(NBLK,),
        in_specs=[
            pl.BlockSpec((RO_BLK, ACC_W), lambda i: (i, 0)),
            pl.BlockSpec((RO_BLK, ACC_W), lambda i: (i, 0)),
            pl.BlockSpec((RO_BLK, HID), lambda i: (i, 0)),
            pl.BlockSpec((RO_BLK, 1), lambda i: (i, 0)),
            pl.BlockSpec((RO_BLK, 1), lambda i: (i, 0)),
            pl.BlockSpec((1, 8), lambda i: (0, 0)),
            pl.BlockSpec((1, HID), lambda i: (0, 0)),
        ],
        out_specs=pl.BlockSpec((RO_BLK, HID), lambda i: (i, 0)),
        out_shape=jax.ShapeDtypeStruct((N, HID), jnp.float32),
    )(acca, accb, hw, asrc, adst, scal, bias_row)


def _readout_body(batch_ref, h_ref, sum_ref, cnt_ref):
    i = pl.program_id(0)

    @pl.when(i == 0)
    def _():
        sum_ref[...] = jnp.zeros_like(sum_ref)
        cnt_ref[...] = jnp.zeros_like(cnt_ref)

    b = batch_ref[0]
    ids = lax.broadcasted_iota(jnp.int32, (B, RO_BLK), 0)
    p = (ids == b).astype(jnp.float32)
    hb = h_ref[...]
    sum_ref[...] += jnp.dot(p, hb, preferred_element_type=jnp.float32)
    cnt_ref[...] += jnp.sum(p, axis=1, keepdims=True)

    @pl.when(i == RO_GRID - 1)
    def _():
        sum_ref[...] = sum_ref[...] / jnp.maximum(cnt_ref[...], 1.0)


def _readout(batch3d, h):
    return pl.pallas_call(
        _readout_body,
        grid=(RO_GRID,),
        in_specs=[
            pl.BlockSpec((1, 1, RO_BLK), lambda i: (i, 0, 0)),
            pl.BlockSpec((RO_BLK, HID), lambda i: (i, 0)),
        ],
        out_specs=[
            pl.BlockSpec((B, HID), lambda i: (0, 0)),
            pl.BlockSpec((B, 1), lambda i: (0, 0)),
        ],
        out_shape=[
            jax.ShapeDtypeStruct((B, HID), jnp.float32),
            jax.ShapeDtypeStruct((B, 1), jnp.float32),
        ],
    )(batch3d, h)


def _block_embed_matrix(tabs, widths):
    total = sum(widths)
    offs = []
    off = 0
    for w in widths:
        offs.append(off)
        off += w
    rows = []
    for k in range(3):
        for tab, w, o in zip(tabs, widths, offs):
            r = jnp.zeros((1, total), jnp.float32)
            r = r.at[0, o:o + w].set(tab[k].astype(jnp.float32))
            rows.append(r)
    return jnp.concatenate(rows, axis=0)


def _leaky(v):
    return jnp.where(v >= 0, v, v * NEG_SLOPE)


def kernel(x, edge_index, edge_attr, batch, smiles_mask, params):
    x3d = x.astype(jnp.int32).reshape(NBLK, RO_BLK, 10)
    e3d = edge_attr.astype(jnp.int32).reshape(EGRID, EBLK, 5)
    src = edge_index[0].astype(jnp.int32)
    dst = edge_index[1].astype(jnp.int32)

    wn = _block_embed_matrix(params['node_tabs'], NODE_W)
    ncnt = _stats_call(x3d, NBLK, RO_BLK, 10)[0]
    pn = ncnt / N
    mean_n = pn @ wn
    var_n = pn @ (wn * wn) - mean_n * mean_n
    scale_n = params['bn_node_g'] / jnp.sqrt(var_n + EPS)
    shift_n = params['bn_node_b'] - mean_n * scale_n

    we = _block_embed_matrix(params['edge_tabs'], EDGE_W)
    ecnt = _stats_call(e3d, EGRID, EBLK, 5)[0]
    pe = ecnt / E
    mean_e = pe @ we
    var_e = pe @ (we * we) - mean_e * mean_e
    scale_e = params['bn_edge_g'] / jnp.sqrt(var_e + EPS)
    shift_e = params['bn_edge_b'] - mean_e * scale_e

    vmat = jnp.stack([p['W_edge'] @ p['att_edge']
                      for p in params['layers']], axis=1)
    lut = we @ (scale_e[:, None] * vmat)
    cconst = shift_e @ vmat
    aer_all, aer_mx = _aer_call(e3d, lut, cconst.reshape(1, LAYERS))
    ael_all = params['bn_edge_b'] @ vmat

    pad = E_PAD - E
    src_p = jnp.concatenate([src, jnp.zeros((pad,), jnp.int32)])
    dst_p = jnp.concatenate([dst, jnp.zeros((pad,), jnp.int32)])
    zeros_rows = jnp.zeros((ROWS_PT, ACC_W), jnp.float32)
    pad_aer = jnp.full((pad,), -1e30, jnp.float32)

    h = None
    for l, p in enumerate(params['layers']):
        a_s, a_d = p['att_src'], p['att_dst']
        if l == 0:
            wf = wn @ (scale_n[:, None] * p['W'])
            c32 = shift_n @ p['W']
            wbig = jnp.concatenate(
                [wf, (wf @ a_s)[:, None], (wf @ a_d)[:, None]], axis=1)
            crow = jnp.concatenate(
                [c32, (c32 @ a_s)[None], (c32 @ a_d)[None]]).reshape(1, 34)
            hw, asrc, adst, mx = _prep_int(x3d, wbig, crow)
        else:
            wbig = jnp.concatenate(
                [p['W'], (p['W'] @ a_s)[:, None], (p['W'] @ a_d)[:, None]],
                axis=1)
            hw, asrc, adst, mx = _prep_f32(
                h, wbig, jnp.zeros((1, 34), jnp.float32))

        ael = ael_all[l]
        m_bound = _leaky(mx[0, 32] + mx[0, 33]
                         + jnp.maximum(aer_mx[0, l], ael))
        aer_p = jnp.concatenate([aer_all[:, l], pad_aer])
        acc = _edge_pass(src_p, dst_p, aer_p,
                         asrc.reshape(N), adst.reshape(N), hw,
                         jnp.full((16,), m_bound, jnp.float32), zeros_rows)
        scal = jnp.stack([m_bound, ael, 0., 0., 0., 0., 0., 0.]).reshape(1, 8)
        h = _finish(acc[:N], acc[N_ACC:N_ACC + N], hw, asrc, adst, scal,
                    p['bias'].reshape(1, HID), with_relu=(l < LAYERS - 1))

    batch3d = batch.astype(jnp.int32).reshape(RO_GRID, 1, RO_BLK)
    pooled, _ = _readout(batch3d, h)
    return (pooled, smiles_mask)


# speedup vs baseline: 1.0402x; 1.0402x over previous
import functools

import jax
import jax.numpy as jnp
from jax import lax
from jax.experimental import pallas as pl
from jax.experimental.pallas import tpu as pltpu
from jax.experimental.pallas import tpu_sc as plsc

N = 50000
E = 800000
B = 512
HID = 32
LAYERS = 5
EPS = 1e-5
NODE_W = [12, 3, 3, 3, 3, 3, 3, 3, 2, 2]
EDGE_W = [8, 3, 2, 2, 2]
NF = 37
EF = 17
NEG_SLOPE = 0.2

NCORES = 2
NSUB = 16
NW = NCORES * NSUB
CH = 128
E_PC = 25088
E_PAD = NW * E_PC
NCHUNK = E_PC // CH
ROWS_PT = 3128
N_ACC = NSUB * ROWS_PT
ACC_W = HID + 1


def _edge_pass_body(src_hbm, dst_hbm, aer_hbm, asrc_hbm, adst_hbm, hw_hbm,
                    m_hbm, zeros_hbm, out_hbm,
                    acc_sh, asrc_sh, adst_sh, srcb, dstb, aerb, abuf, bbuf,
                    tb, rows, stag, mv, sem):
    c = lax.axis_index("c")
    s = lax.axis_index("s")
    wid = s * NCORES + c

    pltpu.sync_copy(zeros_hbm, acc_sh.at[pl.ds(s * ROWS_PT, ROWS_PT)])

    @pl.when(s == 0)
    def _():
        pltpu.sync_copy(asrc_hbm, asrc_sh)
        pltpu.sync_copy(adst_hbm, adst_sh)

    pltpu.sync_copy(m_hbm, mv)
    plsc.subcore_barrier()

    base = wid * E_PC
    mvec = mv[...]

    iota16 = lax.iota(jnp.int32, 16)
    col32 = jnp.full((16,), HID, jnp.int32)

    def chunk(i, carry):
        off = base + i * CH
        pltpu.sync_copy(src_hbm.at[pl.ds(off, CH)], srcb)
        pltpu.sync_copy(dst_hbm.at[pl.ds(off, CH)], dstb)
        pltpu.sync_copy(aer_hbm.at[pl.ds(off, CH)], aerb)
        pltpu.async_copy(asrc_sh.at[srcb], abuf, sem).wait()
        pltpu.async_copy(adst_sh.at[dstb], bbuf, sem).wait()
        pltpu.async_copy(hw_hbm.at[srcb], rows, sem).wait()
        for j in range(CH // 16):
            a = abuf[pl.ds(j * 16, 16)]
            bb = bbuf[pl.ds(j * 16, 16)]
            z = a + bb + aerb[pl.ds(j * 16, 16)]
            z = jnp.where(z >= 0, z, z * NEG_SLOPE)
            t = jnp.exp(z - mvec)
            tb[pl.ds(j * 16, 16)] = t
            plsc.store_scatter(stag, [iota16 + j * 16, col32], t)
            for k in range(16):
                e = j * 16 + k
                ts = plsc.load_gather(tb, [jnp.full((16,), e, jnp.int32)])
                stag[e, pl.ds(0, 16)] = rows[e, pl.ds(0, 16)] * ts
                stag[e, pl.ds(16, 16)] = rows[e, pl.ds(16, 16)] * ts
        pltpu.sync_copy(stag, acc_sh.at[dstb], add=True)
        return carry

    lax.fori_loop(0, NCHUNK, chunk, 0)
    plsc.subcore_barrier()
    pltpu.sync_copy(acc_sh.at[pl.ds(s * ROWS_PT, ROWS_PT)],
                    out_hbm.at[pl.ds(c * N_ACC + s * ROWS_PT, ROWS_PT)])


_edge_pass = functools.partial(
    pl.kernel,
    out_type=jax.ShapeDtypeStruct((NCORES * N_ACC, ACC_W), jnp.float32),
    mesh=plsc.VectorSubcoreMesh(core_axis_name="c", subcore_axis_name="s"),
    compiler_params=pltpu.CompilerParams(needs_layout_passes=False,
                                         use_tc_tiling_on_sc=False),
    scratch_types=[
        pltpu.VMEM_SHARED((N_ACC, ACC_W), jnp.float32),
        pltpu.VMEM_SHARED((N,), jnp.float32),
        pltpu.VMEM_SHARED((N,), jnp.float32),
        pltpu.VMEM((CH,), jnp.int32),
        pltpu.VMEM((CH,), jnp.int32),
        pltpu.VMEM((CH,), jnp.float32),
        pltpu.VMEM((CH,), jnp.float32),
        pltpu.VMEM((CH,), jnp.float32),
        pltpu.VMEM((CH,), jnp.float32),
        pltpu.VMEM((CH, HID), jnp.float32),
        pltpu.VMEM((CH, ACC_W), jnp.float32),
        pltpu.VMEM((16,), jnp.float32),
        pltpu.SemaphoreType.DMA,
    ],
)(_edge_pass_body)


RO_BLK = 5000
RO_GRID = N // RO_BLK
NBLK = N // RO_BLK
EBLK = 8000
EGRID = E // EBLK


def _onehot3(ints):
    return jnp.concatenate(
        [(ints == k).astype(jnp.float32) for k in range(3)], axis=1)


def _stats_body(ints_ref, cnt_ref):
    @pl.when(pl.program_id(0) == 0)
    def _():
        cnt_ref[...] = jnp.zeros_like(cnt_ref)
    oh = _onehot3(ints_ref[0])
    cnt_ref[...] += jnp.sum(oh, axis=0, keepdims=True)


def _stats_call(ints3d, nblk, blk, ncols):
    return pl.pallas_call(
        _stats_body,
        grid=(nblk,),
        in_specs=[pl.BlockSpec((1, blk, ncols), lambda i: (i, 0, 0))],
        out_specs=pl.BlockSpec((1, 3 * ncols), lambda i: (0, 0)),
        out_shape=jax.ShapeDtypeStruct((1, 3 * ncols), jnp.float32),
    )(ints3d)


def _prep_int_body(ints_ref, w_ref, c_ref, hw_ref, asrc_ref, adst_ref, mx_ref):
    @pl.when(pl.program_id(0) == 0)
    def _():
        mx_ref[...] = jnp.full_like(mx_ref, -1e30)
    oh = _onehot3(ints_ref[0])
    out = jnp.dot(oh, w_ref[...], preferred_element_type=jnp.float32) + c_ref[...]
    hw_ref[...] = out[:, :HID]
    asrc_ref[...] = out[:, HID:HID + 1]
    adst_ref[...] = out[:, HID + 1:HID + 2]
    mx_ref[...] = jnp.maximum(mx_ref[...], jnp.max(out, axis=0, keepdims=True))


def _prep_f32_body(h_ref, w_ref, c_ref, hw_ref, asrc_ref, adst_ref, mx_ref):
    @pl.when(pl.program_id(0) == 0)
    def _():
        mx_ref[...] = jnp.full_like(mx_ref, -1e30)
    out = (jnp.dot(h_ref[...], w_ref[...], preferred_element_type=jnp.float32)
           + c_ref[...])
    hw_ref[...] = out[:, :HID]
    asrc_ref[...] = out[:, HID:HID + 1]
    adst_ref[...] = out[:, HID + 1:HID + 2]
    mx_ref[...] = jnp.maximum(mx_ref[...], jnp.max(out, axis=0, keepdims=True))


_PREP_OUT_SPECS = [
    pl.BlockSpec((RO_BLK, HID), lambda i: (i, 0)),
    pl.BlockSpec((RO_BLK, 1), lambda i: (i, 0)),
    pl.BlockSpec((RO_BLK, 1), lambda i: (i, 0)),
    pl.BlockSpec((1, 34), lambda i: (0, 0)),
]
_PREP_OUT_SHAPE = [
    jax.ShapeDtypeStruct((N, HID), jnp.float32),
    jax.ShapeDtypeStruct((N, 1), jnp.float32),
    jax.ShapeDtypeStruct((N, 1), jnp.float32),
    jax.ShapeDtypeStruct((1, 34), jnp.float32),
]


def _prep_int(ints3d, w, crow):
    kdim, odim = w.shape
    return pl.pallas_call(
        _prep_int_body,
        grid=(NBLK,),
        in_specs=[
            pl.BlockSpec((1, RO_BLK, kdim // 3), lambda i: (i, 0, 0)),
            pl.BlockSpec((kdim, odim), lambda i: (0, 0)),
            pl.BlockSpec((1, odim), lambda i: (0, 0)),
        ],
        out_specs=_PREP_OUT_SPECS,
        out_shape=_PREP_OUT_SHAPE,
    )(ints3d, w, crow)


def _prep_f32(h, w, crow):
    kdim, odim = w.shape
    return pl.pallas_call(
        _prep_f32_body,
        grid=(NBLK,),
        in_specs=[
            pl.BlockSpec((RO_BLK, kdim), lambda i: (i, 0)),
            pl.BlockSpec((kdim, odim), lambda i: (0, 0)),
            pl.BlockSpec((1, odim), lambda i: (0, 0)),
        ],
        out_specs=_PREP_OUT_SPECS,
        out_shape=_PREP_OUT_SHAPE,
    )(h, w, crow)


def _aer_body(ints_ref, w_ref, c_ref, out_ref, mx_ref):
    @pl.when(pl.program_id(0) == 0)
    def _():
        mx_ref[...] = jnp.full_like(mx_ref, -1e30)
    oh = _onehot3(ints_ref[0])
    out = jnp.dot(oh, w_ref[...], preferred_element_type=jnp.float32) + c_ref[...]
    out_ref[...] = out
    mx_ref[...] = jnp.maximum(mx_ref[...], jnp.max(out, axis=0, keepdims=True))


def _aer_call(ints3d, w, crow):
    return pl.pallas_call(
        _aer_body,
        grid=(EGRID,),
        in_specs=[
            pl.BlockSpec((1, EBLK, 5), lambda i: (i, 0, 0)),
            pl.BlockSpec((15, LAYERS), lambda i: (0, 0)),
            pl.BlockSpec((1, LAYERS), lambda i: (0, 0)),
        ],
        out_specs=[
            pl.BlockSpec((EBLK, LAYERS), lambda i: (i, 0)),
            pl.BlockSpec((1, LAYERS), lambda i: (0, 0)),
        ],
        out_shape=[
            jax.ShapeDtypeStruct((E, LAYERS), jnp.float32),
            jax.ShapeDtypeStruct((1, LAYERS), jnp.float32),
        ],
    )(ints3d, w, crow)


def _make_finish_body(with_relu):
    def _finish_body(acca_ref, accb_ref, hw_ref, asrc_ref, adst_ref,
                     scal_ref, bias_ref, out_ref):
        asd = asrc_ref[...] + adst_ref[...]
        z = asd + scal_ref[0, 1]
        z = jnp.where(z >= 0, z, z * NEG_SLOPE)
        ts = jnp.exp(z - scal_ref[0, 0])
        num = (acca_ref[:, :HID] + accb_ref[:, :HID]
               + ts * hw_ref[...])
        den = acca_ref[:, HID:HID + 1] + accb_ref[:, HID:HID + 1] + ts
        h = num / den + bias_ref[...]
        out_ref[...] = jnp.maximum(h, 0.0) if with_relu else h
    return _finish_body


def _finish(acca, accb, hw, asrc, adst, scal, bias_row, with_relu):
    return pl.pallas_call(
        _make_finish_body(with_relu),
        grid=(NBLK,),
        in_specs=[
            pl.BlockSpec((RO_BLK, ACC_W), lambda i: (i, 0)),
            pl.BlockSpec((RO_BLK, ACC_W), lambda i: (i, 0)),
            pl.BlockSpec((RO_BLK, HID), lambda i: (i, 0)),
            pl.BlockSpec((RO_BLK, 1), lambda i: (i, 0)),
            pl.BlockSpec((RO_BLK, 1), lambda i: (i, 0)),
            pl.BlockSpec((1, 8), lambda i: (0, 0)),
            pl.BlockSpec((1, HID), lambda i: (0, 0)),
        ],
        out_specs=pl.BlockSpec((RO_BLK, HID), lambda i: (i, 0)),
        out_shape=jax.ShapeDtypeStruct((N, HID), jnp.float32),
    )(acca, accb, hw, asrc, adst, scal, bias_row)


def _readout_body(batch_ref, h_ref, sum_ref, cnt_ref):
    i = pl.program_id(0)

    @pl.when(i == 0)
    def _():
        sum_ref[...] = jnp.zeros_like(sum_ref)
        cnt_ref[...] = jnp.zeros_like(cnt_ref)

    b = batch_ref[0]
    ids = lax.broadcasted_iota(jnp.int32, (B, RO_BLK), 0)
    p = (ids == b).astype(jnp.float32)
    hb = h_ref[...]
    sum_ref[...] += jnp.dot(p, hb, preferred_element_type=jnp.float32)
    cnt_ref[...] += jnp.sum(p, axis=1, keepdims=True)

    @pl.when(i == RO_GRID - 1)
    def _():
        sum_ref[...] = sum_ref[...] / jnp.maximum(cnt_ref[...], 1.0)


def _readout(batch3d, h):
    return pl.pallas_call(
        _readout_body,
        grid=(RO_GRID,),
        in_specs=[
            pl.BlockSpec((1, 1, RO_BLK), lambda i: (i, 0, 0)),
            pl.BlockSpec((RO_BLK, HID), lambda i: (i, 0)),
        ],
        out_specs=[
            pl.BlockSpec((B, HID), lambda i: (0, 0)),
            pl.BlockSpec((B, 1), lambda i: (0, 0)),
        ],
        out_shape=[
            jax.ShapeDtypeStruct((B, HID), jnp.float32),
            jax.ShapeDtypeStruct((B, 1), jnp.float32),
        ],
    )(batch3d, h)


def _block_embed_matrix(tabs, widths):
    total = sum(widths)
    offs = []
    off = 0
    for w in widths:
        offs.append(off)
        off += w
    rows = []
    for k in range(3):
        for tab, w, o in zip(tabs, widths, offs):
            r = jnp.zeros((1, total), jnp.float32)
            r = r.at[0, o:o + w].set(tab[k].astype(jnp.float32))
            rows.append(r)
    return jnp.concatenate(rows, axis=0)


def _leaky(v):
    return jnp.where(v >= 0, v, v * NEG_SLOPE)


def kernel(x, edge_index, edge_attr, batch, smiles_mask, params):
    x3d = x.astype(jnp.int32).reshape(NBLK, RO_BLK, 10)
    e3d = edge_attr.astype(jnp.int32).reshape(EGRID, EBLK, 5)
    src = edge_index[0].astype(jnp.int32)
    dst = edge_index[1].astype(jnp.int32)

    wn = _block_embed_matrix(params['node_tabs'], NODE_W)
    ncnt = _stats_call(x3d, NBLK, RO_BLK, 10)[0]
    pn = ncnt / N
    mean_n = pn @ wn
    var_n = pn @ (wn * wn) - mean_n * mean_n
    scale_n = params['bn_node_g'] / jnp.sqrt(var_n + EPS)
    shift_n = params['bn_node_b'] - mean_n * scale_n

    we = _block_embed_matrix(params['edge_tabs'], EDGE_W)
    ecnt = _stats_call(e3d, EGRID, EBLK, 5)[0]
    pe = ecnt / E
    mean_e = pe @ we
    var_e = pe @ (we * we) - mean_e * mean_e
    scale_e = params['bn_edge_g'] / jnp.sqrt(var_e + EPS)
    shift_e = params['bn_edge_b'] - mean_e * scale_e

    vmat = jnp.stack([p['W_edge'] @ p['att_edge']
                      for p in params['layers']], axis=1)
    lut = we @ (scale_e[:, None] * vmat)
    cconst = shift_e @ vmat
    aer_all, aer_mx = _aer_call(e3d, lut, cconst.reshape(1, LAYERS))
    ael_all = params['bn_edge_b'] @ vmat

    pad = E_PAD - E
    src_p = jnp.concatenate([src, jnp.zeros((pad,), jnp.int32)])
    dst_p = jnp.concatenate([dst, jnp.zeros((pad,), jnp.int32)])
    zeros_rows = jnp.zeros((ROWS_PT, ACC_W), jnp.float32)
    pad_aer = jnp.full((pad,), -1e30, jnp.float32)

    h = None
    for l, p in enumerate(params['layers']):
        a_s, a_d = p['att_src'], p['att_dst']
        if l == 0:
            wf = wn @ (scale_n[:, None] * p['W'])
            c32 = shift_n @ p['W']
            wbig = jnp.concatenate(
                [wf, (wf @ a_s)[:, None], (wf @ a_d)[:, None]], axis=1)
            crow = jnp.concatenate(
                [c32, (c32 @ a_s)[None], (c32 @ a_d)[None]]).reshape(1, 34)
            hw, asrc, adst, mx = _prep_int(x3d, wbig, crow)
        else:
            wbig = jnp.concatenate(
                [p['W'], (p['W'] @ a_s)[:, None], (p['W'] @ a_d)[:, None]],
                axis=1)
            hw, asrc, adst, mx = _prep_f32(
                h, wbig, jnp.zeros((1, 34), jnp.float32))

        ael = ael_all[l]
        m_bound = _leaky(mx[0, 32] + mx[0, 33]
                         + jnp.maximum(aer_mx[0, l], ael))
        aer_p = jnp.concatenate([aer_all[:, l], pad_aer])
        acc = _edge_pass(src_p, dst_p, aer_p,
                         asrc.reshape(N), adst.reshape(N), hw,
                         jnp.full((16,), m_bound, jnp.float32), zeros_rows)
        scal = jnp.stack([m_bound, ael, 0., 0., 0., 0., 0., 0.]).reshape(1, 8)
        h = _finish(acc[:N], acc[N_ACC:N_ACC + N], hw, asrc, adst, scal,
                    p['bias'].reshape(1, HID), with_relu=(l < LAYERS - 1))

    batch3d = batch.astype(jnp.int32).reshape(RO_GRID, 1, RO_BLK)
    pooled, _ = _readout(batch3d, h)
    return (pooled, smiles_mask)
